# baseline (speedup 1.0000x reference)
import functools

import jax
import jax.numpy as jnp
from jax import lax
from jax.experimental import pallas as pl
from jax.experimental.pallas import tpu as pltpu
from jax.experimental.pallas import tpu_sc as plsc

_NC, _NS = 2, 16
_NW = _NC * _NS


def _lrelu(v):
    return jnp.where(v > 0, v, 0.01 * v)


def _sc_mesh():
    return plsc.VectorSubcoreMesh(core_axis_name="c", subcore_axis_name="s",
                                  num_cores=_NC, num_subcores=_NS)


def _sc_gather(table, idx3):
    nw, nchunks, chunk = idx3.shape
    g = nw * nchunks * chunk
    dim = table.shape[1]
    dt = table.dtype
    per_w = nchunks * chunk
    assert nw == _NW

    @functools.partial(
        pl.kernel, mesh=_sc_mesh(),
        out_type=jax.ShapeDtypeStruct((g, dim), dt),
        compiler_params=pltpu.CompilerParams(use_tc_tiling_on_sc=False),
        scratch_types=[pltpu.VMEM((nchunks, chunk), jnp.int32),
                       pltpu.VMEM((chunk, dim), dt),
                       pltpu.VMEM((chunk, dim), dt),
                       pltpu.SemaphoreType.DMA, pltpu.SemaphoreType.DMA,
                       pltpu.SemaphoreType.DMA, pltpu.SemaphoreType.DMA],
    )
    def k(table_hbm, idx_hbm, out_hbm, idx_v, r0, r1, sg0, sg1, so0, so1):
        wid = lax.axis_index("s") * _NC + lax.axis_index("c")
        base = wid * per_w
        pltpu.sync_copy(idx_hbm.at[wid], idx_v)
        rows = (r0, r1)
        gs = (sg0, sg1)
        os = (so0, so1)
        pend_g = {0: pltpu.async_copy(table_hbm.at[idx_v.at[0]], rows[0],
                                      gs[0])}
        pend_o = {}
        for c in range(nchunks):
            b = c & 1
            if c + 1 < nchunks:
                if c >= 1:
                    pend_o.pop(c - 1).wait()
                pend_g[c + 1] = pltpu.async_copy(
                    table_hbm.at[idx_v.at[c + 1]], rows[(c + 1) & 1],
                    gs[(c + 1) & 1])
            pend_g.pop(c).wait()
            pend_o[c] = pltpu.async_copy(
                rows[b], out_hbm.at[pl.ds(base + c * chunk, chunk)], os[b])
        for c in sorted(pend_o):
            pend_o.pop(c).wait()

    return k(table, idx3)


def _sc_scatter_add(vals, idx3, n):
    nw, nchunks, chunk = idx3.shape
    e, dim = vals.shape
    dt = vals.dtype
    per_w = nchunks * chunk
    rows_per_tile = n // _NS
    assert nw == _NW and e == nw * per_w and n % _NS == 0

    @functools.partial(
        pl.kernel, mesh=_sc_mesh(),
        out_type=jax.ShapeDtypeStruct((_NC, n, dim), dt),
        compiler_params=pltpu.CompilerParams(use_tc_tiling_on_sc=False),
        scratch_types=[pltpu.VMEM((nchunks, chunk), jnp.int32),
                       pltpu.VMEM((chunk, dim), dt),
                       pltpu.VMEM((chunk, dim), dt),
                       pltpu.VMEM_SHARED((n, dim), dt),
                       pltpu.SemaphoreType.DMA, pltpu.SemaphoreType.DMA],
    )
    def k(vals_hbm, idx_hbm, zeros_hbm, out_hbm, idx_v, v0, v1, acc_sh,
          s0, s1):
        cid = lax.axis_index("c")
        sid = lax.axis_index("s")
        wid = sid * _NC + cid
        base = wid * per_w
        r0 = sid * rows_per_tile
        pltpu.sync_copy(idx_hbm.at[wid], idx_v)
        pltpu.sync_copy(zeros_hbm.at[pl.ds(r0, rows_per_tile)],
                        acc_sh.at[pl.ds(r0, rows_per_tile)])
        plsc.subcore_barrier()
        bufs = (v0, v1)
        sems = (s0, s1)
        pend = {0: pltpu.async_copy(vals_hbm.at[pl.ds(base, chunk)], bufs[0],
                                    sems[0])}
        for c in range(nchunks):
            b = c & 1
            if c + 1 < nchunks:
                pend[c + 1] = pltpu.async_copy(
                    vals_hbm.at[pl.ds(base + (c + 1) * chunk, chunk)],
                    bufs[(c + 1) & 1], sems[(c + 1) & 1])
            pend.pop(c).wait()
            pltpu.sync_copy(bufs[b], acc_sh.at[idx_v.at[c]], add=True)
        plsc.subcore_barrier()
        pltpu.sync_copy(acc_sh.at[pl.ds(r0, rows_per_tile)],
                        out_hbm.at[cid].at[pl.ds(r0, rows_per_tile)])

    zeros = jnp.zeros((n, dim), dt)
    return k(vals, idx3, zeros)


def _lin0_body(x_ref, w_ref, b_ref, o_ref, ob_ref):
    h0 = _lrelu(
        jnp.dot(x_ref[...], w_ref[...], preferred_element_type=jnp.float32)
        + b_ref[...])
    o_ref[...] = h0
    ob_ref[...] = h0.astype(jnp.bfloat16)


def _encode_nodes(x, w, b):
    n, _ = x.shape
    dim = w.shape[1]
    return pl.pallas_call(
        _lin0_body,
        out_shape=(jax.ShapeDtypeStruct((n, dim), jnp.float32),
                   jax.ShapeDtypeStruct((n, dim), jnp.bfloat16)),
    )(x, w, b.reshape(1, dim))


def _eh_body(ea4_ref, w4_ref, b4_ref, o_ref):
    o_ref[...] = _lrelu(
        jnp.dot(ea4_ref[...], w4_ref[...], preferred_element_type=jnp.float32)
        + b4_ref[...]).astype(jnp.bfloat16)


def _encode_edges(edge_attr4, w4, b4, te4):
    e4, sixteen = edge_attr4.shape
    grid = e4 // te4
    width = w4.shape[1]
    return pl.pallas_call(
        _eh_body,
        grid=(grid,),
        in_specs=[
            pl.BlockSpec((te4, sixteen), lambda i: (i, 0)),
            pl.BlockSpec((sixteen, width), lambda i: (0, 0)),
            pl.BlockSpec((1, width), lambda i: (0, 0)),
        ],
        out_specs=pl.BlockSpec((te4, width), lambda i: (i, 0)),
        out_shape=jax.ShapeDtypeStruct((e4, width), jnp.bfloat16),
    )(edge_attr4, w4, b4)


def _msg_body(u4_ref, eh4_ref, rexp4_ref, w2d4_ref, b24_ref, o4_ref):
    u4 = u4_ref[...]
    g = jnp.dot(u4, w2d4_ref[...], preferred_element_type=jnp.float32)
    ehx = jnp.dot(eh4_ref[...], rexp4_ref[...],
                  preferred_element_type=jnp.float32)
    x = g * ehx
    width = u4.shape[1]
    acc = jnp.dot(u4, b24_ref[...], preferred_element_type=jnp.float32)
    for k in range(x.shape[1] // width):
        acc = acc + x[:, k * width:(k + 1) * width]
    o4_ref[...] = acc


def _messages(u4, eh4, rexp4, w2d4, b24, te4):
    e4, width = u4.shape
    zw = rexp4.shape[1]
    grid = e4 // te4
    return pl.pallas_call(
        _msg_body,
        grid=(grid,),
        in_specs=[
            pl.BlockSpec((te4, width), lambda i: (i, 0)),
            pl.BlockSpec((te4, width), lambda i: (i, 0)),
            pl.BlockSpec((width, zw), lambda i: (0, 0)),
            pl.BlockSpec((width, zw), lambda i: (0, 0)),
            pl.BlockSpec((width, width), lambda i: (0, 0)),
        ],
        out_specs=pl.BlockSpec((te4, width), lambda i: (i, 0)),
        out_shape=jax.ShapeDtypeStruct((e4, width), jnp.float32),
    )(u4, eh4, rexp4, w2d4, b24)


def _gru_body(h_ref, a0_ref, a1_ref, c0_ref, c1_ref,
              convw_ref, convb_ref,
              wih_t_ref, whh_t_ref, bih_ref, bhh_ref, h2_ref, hb_ref, *, dim):
    h = h_ref[...]
    cnt = c0_ref[...].astype(jnp.float32) + c1_ref[...].astype(jnp.float32)
    dinv = 1.0 / jnp.clip(cnt, 1.0, None)
    aggr = (a0_ref[...] + a1_ref[...]) * dinv
    m = _lrelu(
        jnp.dot(h, convw_ref[...], preferred_element_type=jnp.float32)
        + aggr + convb_ref[...])
    gi = jnp.dot(m, wih_t_ref[...], preferred_element_type=jnp.float32) + bih_ref[...]
    gh = jnp.dot(h, whh_t_ref[...], preferred_element_type=jnp.float32) + bhh_ref[...]
    ir, iz, inn = gi[:, :dim], gi[:, dim:2 * dim], gi[:, 2 * dim:]
    hr, hz, hn = gh[:, :dim], gh[:, dim:2 * dim], gh[:, 2 * dim:]
    r = jax.nn.sigmoid(ir + hr)
    z = jax.nn.sigmoid(iz + hz)
    nn = jnp.tanh(inn + r * hn)
    hnew = (1.0 - z) * nn + z * h
    h2_ref[...] = hnew
    hb_ref[...] = hnew.astype(jnp.bfloat16)


def _gru(h, ap, cp, convw, convb, wih_t, whh_t, bih, bhh):
    n, dim = h.shape
    return pl.pallas_call(
        functools.partial(_gru_body, dim=dim),
        out_shape=(jax.ShapeDtypeStruct((n, dim), jnp.float32),
                   jax.ShapeDtypeStruct((n, dim), jnp.bfloat16)),
    )(h, ap[0], ap[1], cp[0], cp[1], convw, convb, wih_t, whh_t,
      bih.reshape(1, -1), bhh.reshape(1, -1))


def _readout_body(gs_ref, gj_ref, s1_ref, s1b_ref, s2_ref, s2b_ref,
                  j1blk_ref, j1b_ref, j2half_ref, j2b_ref,
                  stem_ref, jb_ref):
    hs = _lrelu(
        jnp.dot(gs_ref[...].astype(jnp.float32), s1_ref[...],
                preferred_element_type=jnp.float32)
        + s1b_ref[...])
    stem_ref[...] = (
        jnp.dot(hs, s2_ref[...], preferred_element_type=jnp.float32)
        + s2b_ref[...])
    hj = _lrelu(
        jnp.dot(gj_ref[...].astype(jnp.float32), j1blk_ref[...],
                preferred_element_type=jnp.float32)
        + j1b_ref[...])
    jb_ref[...] = (
        jnp.dot(hj, j2half_ref[...], preferred_element_type=jnp.float32)
        + j2b_ref[...])


def _readouts(gs, gj, s1, s1b, s2, s2b, j1blk, j1b2, j2half, j2b):
    nstem = gs.shape[0]
    njb = gj.shape[0]
    nout = s2.shape[1]
    return pl.pallas_call(
        _readout_body,
        out_shape=(jax.ShapeDtypeStruct((nstem, nout), jnp.float32),
                   jax.ShapeDtypeStruct((njb, 1), jnp.float32)),
    )(gs, gj, s1, s1b.reshape(1, -1), s2, s2b.reshape(1, -1),
      j1blk, j1b2.reshape(1, -1), j2half, j2b.reshape(1, 1))


def _set2set_body(out_ref, batch_ref, gsum_ref, lwq_ref, lwr_ref, lb_ref,
                  so_ref, *, nb, dim):
    out = out_ref[...]
    gates = gsum_ref[...]
    i_ = jax.nn.sigmoid(gates[:, :dim])
    g_ = jnp.tanh(gates[:, 2 * dim:3 * dim])
    o_ = jax.nn.sigmoid(gates[:, 3 * dim:])
    cl = i_ * g_
    qvec = o_ * jnp.tanh(cl)
    e = jnp.dot(out, qvec.T, preferred_element_type=jnp.float32)
    bvec = batch_ref[...]
    iota = jax.lax.broadcasted_iota(jnp.int32, (1, nb), 1)
    oh_bool = bvec == iota
    oh = oh_bool.astype(jnp.float32)
    neg = jnp.float32(-1e30)
    masked = jnp.where(oh_bool, e, neg)
    emax = jnp.max(masked, axis=0, keepdims=True)
    emax = jnp.where(emax < -1e29, 0.0, emax)
    e_shift = e - jnp.dot(oh, emax.T, preferred_element_type=jnp.float32)
    a = jnp.exp(e_shift)
    asum = jnp.sum(oh * a, axis=0, keepdims=True)
    asum = jnp.clip(asum, 1e-12, None)
    a = a / jnp.dot(oh, asum.T, preferred_element_type=jnp.float32)
    ao = a * out
    rvec = jax.lax.dot_general(oh, ao, (((0,), (0,)), ((), ())),
                               preferred_element_type=jnp.float32)
    so_ref[...] = (
        jnp.dot(rvec, lwr_ref[...], preferred_element_type=jnp.float32)
        + jnp.dot(qvec, lwq_ref[...], preferred_element_type=jnp.float32)
        + lb_ref[...])


def _set2set(out, batch, gsum, lwq, lwr, lb, nb):
    n, dim = out.shape
    return pl.pallas_call(
        functools.partial(_set2set_body, nb=nb, dim=dim),
        out_shape=jax.ShapeDtypeStruct((nb, 2), jnp.float32),
    )(out, batch.reshape(n, 1), gsum.reshape(1, -1), lwq, lwr,
      lb.reshape(1, 2))


def kernel(x, edge_attr, params, edge_index, stem_atmidx, jbond_atmidx, batch):
    p = params
    n, _ = x.shape
    e = edge_attr.shape[0]
    dim = p['lin0_w'].shape[1]
    nb = 256

    src = edge_index[0]
    dst = edge_index[1]

    eye4 = jnp.eye(4, dtype=jnp.float32)
    b2 = p['en2_b'].reshape(dim, dim)
    arr = p['en2_w'].reshape(dim, dim, dim)
    w2d4 = jnp.einsum('rs,kdf->rdksf', eye4, arr).reshape(
        4 * dim, 4 * dim * dim).astype(jnp.bfloat16)
    rexp4 = (eye4[:, None, None, :, None]
             * jnp.eye(dim, dtype=jnp.float32)[None, :, :, None, None]
             * jnp.ones((1, 1, 1, 1, dim), jnp.float32)
             ).reshape(4 * dim, 4 * dim * dim).astype(jnp.bfloat16)
    b24 = jnp.kron(eye4, b2).astype(jnp.bfloat16)
    w4e = jnp.kron(eye4, p['en1_w'])
    b4e = jnp.tile(p['en1_b'], 4).reshape(1, 4 * dim)
    wih_t = p['gru_wih'].T
    whh_t = p['gru_whh'].T
    j1 = p['j1_w']
    j1blk = jnp.block([[j1, jnp.zeros_like(j1)], [jnp.zeros_like(j1), j1]])
    j1b2 = jnp.concatenate([p['j1_b'], p['j1_b']])
    j2half = 0.5 * jnp.concatenate([p['j2_w'], p['j2_w']], axis=0)
    lwq = p['lout_w'][:dim]
    lwr = p['lout_w'][dim:]
    gsum = p['lstm_bih'] + p['lstm_bhh']

    te4 = 1000 if (e // 4) % 1000 == 0 else e // 4
    chunk = 1000
    src3 = src.reshape(_NW, e // (_NW * chunk), chunk)
    dst3 = dst.reshape(_NW, e // (_NW * chunk), chunk)

    h, hb = _encode_nodes(x, p['lin0_w'], p['lin0_b'])
    eh4 = _encode_edges(edge_attr.reshape(e // 4, 16), w4e, b4e, e // 32)

    cp = _sc_scatter_add(jnp.ones((e, dim), jnp.bfloat16), dst3, n)

    for _ in range(6):
        u = _sc_gather(hb, src3)
        msg4 = _messages(u.reshape(e // 4, 4 * dim), eh4, rexp4, w2d4,
                         b24, te4)
        ap = _sc_scatter_add(msg4.reshape(e, dim), dst3, n)
        h, hb = _gru(h, ap, cp, p['conv_root'],
                     p['conv_b'].reshape(1, dim), wih_t, whh_t,
                     p['gru_bih'], p['gru_bhh'])
    out = h

    fidx = jnp.concatenate([stem_atmidx, jbond_atmidx.reshape(-1)])
    gth = _sc_gather(hb, fidx.reshape(_NW, 1, fidx.shape[0] // _NW))
    gs = gth[:stem_atmidx.shape[0]]
    gj = gth[stem_atmidx.shape[0]:].reshape(-1, 2 * dim)

    stem_preds, jb = _readouts(gs, gj, p['s1_w'], p['s1_b'], p['s2_w'],
                               p['s2_b'], j1blk, j1b2, j2half, p['j2_b'])
    jbond_preds = jb.reshape(-1)

    scalar_outs = _set2set(out, batch, gsum, lwq, lwr, p['lout_b'], nb)
    return scalar_outs, stem_preds, jbond_preds


# speedup vs baseline: 1.1379x; 1.1379x over previous
import functools

import jax
import jax.numpy as jnp
from jax import lax
from jax.experimental import pallas as pl
from jax.experimental.pallas import tpu as pltpu
from jax.experimental.pallas import tpu_sc as plsc

_NC, _NS = 2, 16
_NW = _NC * _NS


def _lrelu(v):
    return jnp.where(v > 0, v, 0.01 * v)


def _sc_mesh():
    return plsc.VectorSubcoreMesh(core_axis_name="c", subcore_axis_name="s",
                                  num_cores=_NC, num_subcores=_NS)


def _sc_gather(table, idx3):
    nw, nchunks, chunk = idx3.shape
    g = nw * nchunks * chunk
    dim = table.shape[1]
    dt = table.dtype
    per_w = nchunks * chunk
    assert nw == _NW

    @functools.partial(
        pl.kernel, mesh=_sc_mesh(),
        out_type=jax.ShapeDtypeStruct((g, dim), dt),
        compiler_params=pltpu.CompilerParams(use_tc_tiling_on_sc=False),
        scratch_types=[pltpu.VMEM((nchunks, chunk), jnp.int32),
                       pltpu.VMEM((chunk, dim), dt),
                       pltpu.VMEM((chunk, dim), dt),
                       pltpu.SemaphoreType.DMA, pltpu.SemaphoreType.DMA,
                       pltpu.SemaphoreType.DMA, pltpu.SemaphoreType.DMA],
    )
    def k(table_hbm, idx_hbm, out_hbm, idx_v, r0, r1, sg0, sg1, so0, so1):
        wid = lax.axis_index("s") * _NC + lax.axis_index("c")
        base = wid * per_w
        pltpu.sync_copy(idx_hbm.at[wid], idx_v)
        rows = (r0, r1)
        gs = (sg0, sg1)
        os = (so0, so1)
        pend_g = {0: pltpu.async_copy(table_hbm.at[idx_v.at[0]], rows[0],
                                      gs[0])}
        pend_o = {}
        for c in range(nchunks):
            b = c & 1
            if c + 1 < nchunks:
                if c >= 1:
                    pend_o.pop(c - 1).wait()
                pend_g[c + 1] = pltpu.async_copy(
                    table_hbm.at[idx_v.at[c + 1]], rows[(c + 1) & 1],
                    gs[(c + 1) & 1])
            pend_g.pop(c).wait()
            pend_o[c] = pltpu.async_copy(
                rows[b], out_hbm.at[pl.ds(base + c * chunk, chunk)], os[b])
        for c in sorted(pend_o):
            pend_o.pop(c).wait()

    return k(table, idx3)


def _sc_scatter_add(vals, idx3, n):
    nw, nchunks, chunk = idx3.shape
    e, dim = vals.shape
    dt = vals.dtype
    per_w = nchunks * chunk
    rows_per_tile = n // _NS
    assert nw == _NW and e == nw * per_w and n % _NS == 0

    @functools.partial(
        pl.kernel, mesh=_sc_mesh(),
        out_type=jax.ShapeDtypeStruct((_NC, n, dim), dt),
        compiler_params=pltpu.CompilerParams(use_tc_tiling_on_sc=False),
        scratch_types=[pltpu.VMEM((nchunks, chunk), jnp.int32),
                       pltpu.VMEM((chunk, dim), dt),
                       pltpu.VMEM((chunk, dim), dt),
                       pltpu.VMEM_SHARED((n, dim), dt),
                       pltpu.SemaphoreType.DMA, pltpu.SemaphoreType.DMA],
    )
    def k(vals_hbm, idx_hbm, zeros_hbm, out_hbm, idx_v, v0, v1, acc_sh,
          s0, s1):
        cid = lax.axis_index("c")
        sid = lax.axis_index("s")
        wid = sid * _NC + cid
        base = wid * per_w
        r0 = sid * rows_per_tile
        pltpu.sync_copy(idx_hbm.at[wid], idx_v)
        pltpu.sync_copy(zeros_hbm.at[pl.ds(r0, rows_per_tile)],
                        acc_sh.at[pl.ds(r0, rows_per_tile)])
        plsc.subcore_barrier()
        bufs = (v0, v1)
        sems = (s0, s1)
        pend = {0: pltpu.async_copy(vals_hbm.at[pl.ds(base, chunk)], bufs[0],
                                    sems[0])}
        for c in range(nchunks):
            b = c & 1
            if c + 1 < nchunks:
                pend[c + 1] = pltpu.async_copy(
                    vals_hbm.at[pl.ds(base + (c + 1) * chunk, chunk)],
                    bufs[(c + 1) & 1], sems[(c + 1) & 1])
            pend.pop(c).wait()
            pltpu.sync_copy(bufs[b], acc_sh.at[idx_v.at[c]], add=True)
        plsc.subcore_barrier()
        pltpu.sync_copy(acc_sh.at[pl.ds(r0, rows_per_tile)],
                        out_hbm.at[cid].at[pl.ds(r0, rows_per_tile)])

    zeros = jnp.zeros((n, dim), dt)
    return k(vals, idx3, zeros)


def _lin0_body(x_ref, w_ref, b_ref, o_ref):
    o_ref[...] = _lrelu(
        jnp.dot(x_ref[...], w_ref[...], preferred_element_type=jnp.float32)
        + b_ref[...])


def _encode_nodes(x, w, b):
    n, _ = x.shape
    dim = w.shape[1]
    return pl.pallas_call(
        _lin0_body,
        out_shape=jax.ShapeDtypeStruct((n, dim), jnp.float32),
    )(x, w, b.reshape(1, dim))


def _eh_body(ea4_ref, w4_ref, b4_ref, o_ref):
    o_ref[...] = _lrelu(
        jnp.dot(ea4_ref[...], w4_ref[...], preferred_element_type=jnp.float32)
        + b4_ref[...])


def _encode_edges(edge_attr4, w4, b4, te4):
    e4, sixteen = edge_attr4.shape
    grid = e4 // te4
    width = w4.shape[1]
    return pl.pallas_call(
        _eh_body,
        grid=(grid,),
        in_specs=[
            pl.BlockSpec((te4, sixteen), lambda i: (i, 0)),
            pl.BlockSpec((sixteen, width), lambda i: (0, 0)),
            pl.BlockSpec((1, width), lambda i: (0, 0)),
        ],
        out_specs=pl.BlockSpec((te4, width), lambda i: (i, 0)),
        out_shape=jax.ShapeDtypeStruct((e4, width), jnp.float32),
    )(edge_attr4, w4, b4)


def _msg_body(u4_ref, eh4_ref, rexp4_ref, w2d4_ref, b24_ref, o4_ref):
    u4 = u4_ref[...].astype(jnp.bfloat16)
    g = jnp.dot(u4, w2d4_ref[...], preferred_element_type=jnp.float32)
    ehx = jnp.dot(eh4_ref[...].astype(jnp.bfloat16), rexp4_ref[...],
                  preferred_element_type=jnp.float32)
    x = g * ehx
    width = u4.shape[1]
    acc = jnp.dot(u4, b24_ref[...], preferred_element_type=jnp.float32)
    for k in range(x.shape[1] // width):
        acc = acc + x[:, k * width:(k + 1) * width]
    o4_ref[...] = acc


def _messages(u4, eh4, rexp4, w2d4, b24, te4):
    e4, width = u4.shape
    zw = rexp4.shape[1]
    grid = e4 // te4
    return pl.pallas_call(
        _msg_body,
        grid=(grid,),
        in_specs=[
            pl.BlockSpec((te4, width), lambda i: (i, 0)),
            pl.BlockSpec((te4, width), lambda i: (i, 0)),
            pl.BlockSpec((width, zw), lambda i: (0, 0)),
            pl.BlockSpec((width, zw), lambda i: (0, 0)),
            pl.BlockSpec((width, width), lambda i: (0, 0)),
        ],
        out_specs=pl.BlockSpec((te4, width), lambda i: (i, 0)),
        out_shape=jax.ShapeDtypeStruct((e4, width), jnp.float32),
    )(u4, eh4, rexp4, w2d4, b24)


def _gru_body(h_ref, a0_ref, a1_ref, c0_ref, c1_ref,
              convw_ref, convb_ref,
              wih_t_ref, whh_t_ref, bih_ref, bhh_ref, h2_ref, *, dim):
    h = h_ref[...]
    cnt = c0_ref[...].astype(jnp.float32) + c1_ref[...].astype(jnp.float32)
    dinv = 1.0 / jnp.clip(cnt, 1.0, None)
    aggr = (a0_ref[...] + a1_ref[...]) * dinv
    m = _lrelu(
        jnp.dot(h, convw_ref[...], preferred_element_type=jnp.float32)
        + aggr + convb_ref[...])
    gi = jnp.dot(m, wih_t_ref[...], preferred_element_type=jnp.float32) + bih_ref[...]
    gh = jnp.dot(h, whh_t_ref[...], preferred_element_type=jnp.float32) + bhh_ref[...]
    ir, iz, inn = gi[:, :dim], gi[:, dim:2 * dim], gi[:, 2 * dim:]
    hr, hz, hn = gh[:, :dim], gh[:, dim:2 * dim], gh[:, 2 * dim:]
    r = jax.nn.sigmoid(ir + hr)
    z = jax.nn.sigmoid(iz + hz)
    nn = jnp.tanh(inn + r * hn)
    hnew = (1.0 - z) * nn + z * h
    h2_ref[...] = hnew


def _gru(h, ap, cp, convw, convb, wih_t, whh_t, bih, bhh):
    n, dim = h.shape
    return pl.pallas_call(
        functools.partial(_gru_body, dim=dim),
        out_shape=jax.ShapeDtypeStruct((n, dim), jnp.float32),
    )(h, ap[0], ap[1], cp[0], cp[1], convw, convb, wih_t, whh_t,
      bih.reshape(1, -1), bhh.reshape(1, -1))


def _readout_body(gs_ref, gj_ref, s1_ref, s1b_ref, s2_ref, s2b_ref,
                  j1blk_ref, j1b_ref, j2half_ref, j2b_ref,
                  stem_ref, jb_ref):
    hs = _lrelu(
        jnp.dot(gs_ref[...].astype(jnp.float32), s1_ref[...],
                preferred_element_type=jnp.float32)
        + s1b_ref[...])
    stem_ref[...] = (
        jnp.dot(hs, s2_ref[...], preferred_element_type=jnp.float32)
        + s2b_ref[...])
    hj = _lrelu(
        jnp.dot(gj_ref[...].astype(jnp.float32), j1blk_ref[...],
                preferred_element_type=jnp.float32)
        + j1b_ref[...])
    jb_ref[...] = (
        jnp.dot(hj, j2half_ref[...], preferred_element_type=jnp.float32)
        + j2b_ref[...])


def _readouts(gs, gj, s1, s1b, s2, s2b, j1blk, j1b2, j2half, j2b):
    nstem = gs.shape[0]
    njb = gj.shape[0]
    nout = s2.shape[1]
    return pl.pallas_call(
        _readout_body,
        out_shape=(jax.ShapeDtypeStruct((nstem, nout), jnp.float32),
                   jax.ShapeDtypeStruct((njb, 1), jnp.float32)),
    )(gs, gj, s1, s1b.reshape(1, -1), s2, s2b.reshape(1, -1),
      j1blk, j1b2.reshape(1, -1), j2half, j2b.reshape(1, 1))


def _set2set_body(out_ref, batch_ref, gsum_ref, lwq_ref, lwr_ref, lb_ref,
                  so_ref, *, nb, dim):
    out = out_ref[...]
    gates = gsum_ref[...]
    i_ = jax.nn.sigmoid(gates[:, :dim])
    g_ = jnp.tanh(gates[:, 2 * dim:3 * dim])
    o_ = jax.nn.sigmoid(gates[:, 3 * dim:])
    cl = i_ * g_
    qvec = o_ * jnp.tanh(cl)
    e = jnp.dot(out, qvec.T, preferred_element_type=jnp.float32)
    bvec = batch_ref[...]
    iota = jax.lax.broadcasted_iota(jnp.int32, (1, nb), 1)
    oh_bool = bvec == iota
    oh = oh_bool.astype(jnp.float32)
    neg = jnp.float32(-1e30)
    masked = jnp.where(oh_bool, e, neg)
    emax = jnp.max(masked, axis=0, keepdims=True)
    emax = jnp.where(emax < -1e29, 0.0, emax)
    e_shift = e - jnp.dot(oh, emax.T, preferred_element_type=jnp.float32)
    a = jnp.exp(e_shift)
    asum = jnp.sum(oh * a, axis=0, keepdims=True)
    asum = jnp.clip(asum, 1e-12, None)
    a = a / jnp.dot(oh, asum.T, preferred_element_type=jnp.float32)
    ao = a * out
    rvec = jax.lax.dot_general(oh, ao, (((0,), (0,)), ((), ())),
                               preferred_element_type=jnp.float32)
    so_ref[...] = (
        jnp.dot(rvec, lwr_ref[...], preferred_element_type=jnp.float32)
        + jnp.dot(qvec, lwq_ref[...], preferred_element_type=jnp.float32)
        + lb_ref[...])


def _set2set(out, batch, gsum, lwq, lwr, lb, nb):
    n, dim = out.shape
    return pl.pallas_call(
        functools.partial(_set2set_body, nb=nb, dim=dim),
        out_shape=jax.ShapeDtypeStruct((nb, 2), jnp.float32),
    )(out, batch.reshape(n, 1), gsum.reshape(1, -1), lwq, lwr,
      lb.reshape(1, 2))


def kernel(x, edge_attr, params, edge_index, stem_atmidx, jbond_atmidx, batch):
    p = params
    n, _ = x.shape
    e = edge_attr.shape[0]
    dim = p['lin0_w'].shape[1]
    nb = 256

    src = edge_index[0]
    dst = edge_index[1]

    eye4 = jnp.eye(4, dtype=jnp.float32)
    b2 = p['en2_b'].reshape(dim, dim)
    arr = p['en2_w'].reshape(dim, dim, dim)
    w2d4 = jnp.einsum('rs,kdf->rdksf', eye4, arr).reshape(
        4 * dim, 4 * dim * dim).astype(jnp.bfloat16)
    rexp4 = (eye4[:, None, None, :, None]
             * jnp.eye(dim, dtype=jnp.float32)[None, :, :, None, None]
             * jnp.ones((1, 1, 1, 1, dim), jnp.float32)
             ).reshape(4 * dim, 4 * dim * dim).astype(jnp.bfloat16)
    b24 = jnp.kron(eye4, b2).astype(jnp.bfloat16)
    w4e = jnp.kron(eye4, p['en1_w'])
    b4e = jnp.tile(p['en1_b'], 4).reshape(1, 4 * dim)
    wih_t = p['gru_wih'].T
    whh_t = p['gru_whh'].T
    j1 = p['j1_w']
    j1blk = jnp.block([[j1, jnp.zeros_like(j1)], [jnp.zeros_like(j1), j1]])
    j1b2 = jnp.concatenate([p['j1_b'], p['j1_b']])
    j2half = 0.5 * jnp.concatenate([p['j2_w'], p['j2_w']], axis=0)
    lwq = p['lout_w'][:dim]
    lwr = p['lout_w'][dim:]
    gsum = p['lstm_bih'] + p['lstm_bhh']

    te4 = 1000 if (e // 4) % 1000 == 0 else e // 4
    chunk = 1000
    src3 = src.reshape(_NW, e // (_NW * chunk), chunk)
    dst3 = dst.reshape(_NW, e // (_NW * chunk), chunk)

    h = _encode_nodes(x, p['lin0_w'], p['lin0_b'])
    eh4 = _encode_edges(edge_attr.reshape(e // 4, 16), w4e, b4e, e // 32)

    cp = _sc_scatter_add(jnp.ones((e, dim), jnp.bfloat16), dst3, n)

    for _ in range(6):
        u = _sc_gather(h, src3)
        msg4 = _messages(u.reshape(e // 4, 4 * dim), eh4, rexp4, w2d4,
                         b24, te4)
        ap = _sc_scatter_add(msg4.reshape(e, dim), dst3, n)
        h = _gru(h, ap, cp, p['conv_root'],
                 p['conv_b'].reshape(1, dim), wih_t, whh_t,
                 p['gru_bih'], p['gru_bhh'])
    out = h

    fidx = jnp.concatenate([stem_atmidx, jbond_atmidx.reshape(-1)])
    gth = _sc_gather(out, fidx.reshape(_NW, 1, fidx.shape[0] // _NW))
    gs = gth[:stem_atmidx.shape[0]]
    gj = gth[stem_atmidx.shape[0]:].reshape(-1, 2 * dim)

    stem_preds, jb = _readouts(gs, gj, p['s1_w'], p['s1_b'], p['s2_w'],
                               p['s2_b'], j1blk, j1b2, j2half, p['j2_b'])
    jbond_preds = jb.reshape(-1)

    scalar_outs = _set2set(out, batch, gsum, lwq, lwr, p['lout_b'], nb)
    return scalar_outs, stem_preds, jbond_preds
